# baseline (speedup 1.0000x reference)
import functools

import jax
import jax.numpy as jnp
from jax import lax
from jax.experimental import pallas as pl
from jax.experimental.pallas import tpu as pltpu
from jax.experimental.pallas import tpu_sc as plsc

_N = 10000
_E = 320000
_K = 128
_NCH = _E // _K
_NS = 16
_NC = 2
_RPT = 632
_RLAST = _N - _RPT * (_NS - 1)


def _per_tile_rows(s, fn):
    @pl.when(s < _NS - 1)
    def _():
        fn(s * _RPT, _RPT)

    @pl.when(s == _NS - 1)
    def _():
        fn(_RPT * (_NS - 1), _RLAST)


def _lr(x):
    return jnp.where(x >= 0, x, 0.01 * x)


def _sc_mesh():
    return plsc.VectorSubcoreMesh(
        core_axis_name="c", subcore_axis_name="s",
        num_cores=_NC, num_subcores=_NS)


def _sc_degree(col, zeros):
    ones = jnp.ones((_K, 128), jnp.float32)

    @functools.partial(
        pl.kernel,
        out_type=jax.ShapeDtypeStruct((_NC, _N, 128), jnp.float32),
        mesh=_sc_mesh(),
        scratch_types=[
            pltpu.VMEM_SHARED((_N, 128), jnp.float32),
            pltpu.VMEM((_K,), jnp.int32),
            pltpu.VMEM((_K, 128), jnp.float32),
            pltpu.SemaphoreType.DMA,
        ],
    )
    def k(col_hbm, ones_hbm, zeros_hbm, cnt_out, acc, colbuf, onesbuf, sem):
        c = lax.axis_index("c")
        s = lax.axis_index("s")
        w = c * _NS + s
        _per_tile_rows(s, lambda base, sz: pltpu.sync_copy(
            zeros_hbm.at[pl.ds(base, sz)], acc.at[pl.ds(base, sz)]))
        pltpu.sync_copy(ones_hbm, onesbuf)
        plsc.subcore_barrier()
        nloc = 78 + (w < 4).astype(jnp.int32)

        def step(i, carry):
            base = (w + i * (_NC * _NS)) * _K
            pltpu.sync_copy(col_hbm.at[pl.ds(base, _K)], colbuf)
            pltpu.sync_copy(onesbuf, acc.at[colbuf], add=True)
            return carry

        lax.fori_loop(0, nloc, step, 0)
        plsc.subcore_barrier()
        _per_tile_rows(s, lambda base, sz: pltpu.sync_copy(
            acc.at[pl.ds(base, sz)], cnt_out.at[c, pl.ds(base, sz)]))

    return k(col, ones, zeros)


def _sc_convs(row, col, convs, zeros):
    ins = []
    out_type = []
    for cv in convs:
        if cv[0] == 'col':
            ins += [cv[1], cv[2]]
            out_type += [jax.ShapeDtypeStruct((_N, 128), jnp.float32)] * 2
        else:
            ins.append(cv[1])
            out_type.append(jax.ShapeDtypeStruct((_NC, _N, 128), jnp.float32))
    n_in = len(ins)
    scratch = [pltpu.VMEM_SHARED((_N, 128), jnp.float32)]
    scratch += [pltpu.VMEM((_K, 128), jnp.float32) for _ in range(2)]
    scratch += [pltpu.VMEM((_K,), jnp.int32) for _ in range(8)]
    scratch += [pltpu.SemaphoreType.DMA for _ in range(8)]

    @functools.partial(
        pl.kernel, out_type=out_type, mesh=_sc_mesh(),
        scratch_types=scratch,
    )
    def k(row_hbm, col_hbm, zeros_hbm, *rest):
        zs = rest[:n_in]
        outs = rest[n_in:2 * n_in]
        sc = rest[2 * n_in:]
        acc = sc[0]
        gb = sc[1:3]
        rb = sc[3:7]
        cb = sc[7:11]
        gsem = sc[11:13]
        ssem = sc[13:15]
        isem = sc[15:19]
        c = lax.axis_index("c")
        s = lax.axis_index("s")

        def edge_sweep(z, chunk0, npipe, nepi):
            def base_of(i):
                return (chunk0 + s + jnp.minimum(i, npipe - 1) * _NS) * _K

            def issue_idx(i, q):
                pltpu.async_copy(row_hbm.at[pl.ds(base_of(i), _K)],
                                 rb[q], isem[q])
                pltpu.async_copy(col_hbm.at[pl.ds(base_of(i), _K)],
                                 cb[q], isem[q])

            def wait_idx(q):
                pltpu.make_async_copy(row_hbm.at[pl.ds(0, _K)],
                                      rb[q], isem[q]).wait()
                pltpu.make_async_copy(col_hbm.at[pl.ds(0, _K)],
                                      cb[q], isem[q]).wait()

            def issue_gather(x, q):
                pltpu.async_copy(z.at[rb[q]], gb[x], gsem[x])

            def wait_gather(x):
                pltpu.make_async_copy(zeros_hbm.at[pl.ds(0, _K)],
                                      gb[x], gsem[x]).wait()

            def issue_scatter(x, q):
                pltpu.async_copy(gb[x], acc.at[cb[q]], ssem[x], add=True)

            def wait_scatter(x):
                pltpu.make_async_copy(zeros_hbm.at[pl.ds(0, _K)],
                                      gb[x], ssem[x]).wait()

            def half(i, p2, p4, first=False):
                wait_gather(p2)
                issue_scatter(p2, p4)
                if not first:
                    wait_scatter(1 - p2)
                wait_idx((p4 + 1) % 4)
                issue_gather(1 - p2, (p4 + 1) % 4)
                issue_idx(i + 2, (p4 + 2) % 4)

            issue_idx(0, 0)
            issue_idx(1, 1)
            wait_idx(0)
            issue_gather(0, 0)
            half(0, 0, 0, first=True)

            w_loop = (npipe - 1) // 4
            rem = npipe - 1 - 4 * w_loop

            def body(u, carry):
                i0 = 4 * u + 1
                for kk in range(4):
                    half(i0 + kk, (1 + kk) % 2, (1 + kk) % 4)
                return carry

            lax.fori_loop(0, w_loop, body, 0)
            for kk in range(rem):
                half(4 * w_loop + 1 + kk, (1 + kk) % 2, (1 + kk) % 4)
            wait_scatter((npipe - 1) % 2)
            wait_gather(npipe % 2)
            wait_idx((npipe + 1) % 4)

            def step(i, carry):
                base = (chunk0 + s + i * _NS) * _K
                pltpu.sync_copy(row_hbm.at[pl.ds(base, _K)], rb[0])
                pltpu.sync_copy(col_hbm.at[pl.ds(base, _K)], cb[0])
                pltpu.async_copy(z.at[rb[0]], gb[0], gsem[0]).wait()
                pltpu.sync_copy(gb[0], acc.at[cb[0]], add=True)
                return carry

            lax.fori_loop(npipe, npipe + nepi, step, 0)

        def init_from(src):
            _per_tile_rows(s, lambda base, sz: pltpu.sync_copy(
                src.at[pl.ds(base, sz)], acc.at[pl.ds(base, sz)]))

        zi = 0
        oi = 0
        for cv in convs:
            if cv[0] == 'col':
                z_lo, z_hi = zs[zi], zs[zi + 1]
                o_lo, o_hi = outs[oi], outs[oi + 1]
                zi += 2
                oi += 2
                nepi = (s < 4).astype(jnp.int32)
                for half, (z, o) in enumerate(((z_lo, o_lo), (z_hi, o_hi))):
                    @pl.when(c == half)
                    def _(z=z, o=o, nepi=nepi):
                        init_from(z)
                        plsc.subcore_barrier()
                        edge_sweep(z, 0, 156, nepi)
                        plsc.subcore_barrier()
                        _per_tile_rows(s, lambda base, sz: pltpu.sync_copy(
                            acc.at[pl.ds(base, sz)], o.at[pl.ds(base, sz)]))
                        plsc.subcore_barrier()
            else:
                z = zs[zi]
                o = outs[oi]
                zi += 1
                oi += 1

                @pl.when(c == 0)
                def _(z=z):
                    init_from(z)

                @pl.when(c == 1)
                def _():
                    init_from(zeros_hbm)

                plsc.subcore_barrier()
                nepi = (s < 2).astype(jnp.int32)
                edge_sweep(z, c * (_NCH // _NC), 78, nepi)
                plsc.subcore_barrier()
                _per_tile_rows(s, lambda base, sz: pltpu.sync_copy(
                    acc.at[pl.ds(base, sz)], o.at[c, pl.ds(base, sz)]))
                plsc.subcore_barrier()

    return k(row, col, zeros, *ins)


_B = 1000
_G = _N // _B


def _row_spec(d):
    return pl.BlockSpec((_B, d), lambda i: (i, 0))


def _part_spec():
    return pl.BlockSpec((_NC, _B, 128), lambda i: (0, i, 0))


def _full_spec(shape):
    nd = len(shape)
    return pl.BlockSpec(shape, lambda i: (0,) * nd)


def _tc_call(body, ins, in_full, outs):
    in_specs = []
    args = []
    for a, d in ins:
        args.append(a)
        if d == 'part':
            in_specs.append(_part_spec())
        elif d is not None:
            in_specs.append(_row_spec(d))
        else:
            in_specs.append(_full_spec(a.shape))
    out_specs = [_row_spec(d) for d in outs]
    out_shape = [jax.ShapeDtypeStruct((_N, d), jnp.float32) for d in outs]
    return pl.pallas_call(
        body,
        grid=(_G,),
        in_specs=in_specs,
        out_specs=out_specs,
        out_shape=out_shape,
    )(*args)


def _dot(a, b):
    return jnp.dot(a, b, preferred_element_type=jnp.float32)


def kernel(discrete_x, continous_x, edge_index, edge_attr, churn_date, params):
    p = params
    row = edge_index[0]
    col = edge_index[1]
    x = discrete_x

    cpad = jnp.pad(churn_date, ((0, 0), (0, 112)))
    wns0 = jnp.pad(p['Wns0'], ((0, 112), (0, 0)))

    def r1(b):
        return b.reshape(1, -1)

    zeros = jnp.zeros((_N, 128), jnp.float32)
    cnt = _sc_degree(col, zeros)

    def t1(x_ref, c_ref, wd, bd, wg0, bg0, wnf0, bnf0, wns0_, bns0,
           xd_o, xg0_o, xf0_o, xns0_o):
        xb = x_ref[...]
        cb = c_ref[...]
        xd_o[...] = _lr(_dot(xb, wd[...]) + bd[...])
        xg0_o[...] = _lr(_dot(xb, wg0[...]) + bg0[...])
        xf0_o[...] = _lr(_dot(xb, wnf0[...]) + bnf0[...])
        xns0_o[...] = _lr(_dot(cb, wns0_[...]) + bns0[...])

    xd, xg0, xf0, xns0 = _tc_call(
        t1,
        [(x, 128), (cpad, 128),
         (p['Wd'], None), (r1(p['bd']), None),
         (p['Wg0'], None), (r1(p['bg0']), None),
         (p['Wnf0'], None), (r1(p['bnf0']), None),
         (wns0, None), (r1(p['bns0']), None)],
        None, [128, 128, 256, 128])

    def t2(cnt_ref, xg0_ref, xf0_ref, xns0_ref, wg1, wnf1, wns1,
           dinv_o, zg_o, zfa_o, zfb_o, znsa_o, znsb_o):
        cb = cnt_ref[...]
        deg = cb[0, :, 0:1] + cb[1, :, 0:1] + 1.0
        dinv = lax.rsqrt(deg)
        dinv_o[...] = dinv
        zg_o[...] = dinv * _dot(xg0_ref[...], wg1[...])
        zf = dinv * _dot(xf0_ref[...], wnf1[...])
        zns = dinv * _dot(xns0_ref[...], wns1[...])
        zfa_o[...] = zf[:, :128]
        zfb_o[...] = zf[:, 128:]
        znsa_o[...] = zns[:, :128]
        znsb_o[...] = zns[:, 128:]

    t2_out = [jax.ShapeDtypeStruct((_N, d), jnp.float32)
              for d in (1, 128, 128, 128, 128, 128)]
    dinv, zg, zfa, zfb, znsa, znsb = pl.pallas_call(
        t2,
        grid=(_G,),
        in_specs=[_part_spec(),
                  _row_spec(128), _row_spec(256), _row_spec(128),
                  _full_spec(p['Wg1'].shape), _full_spec(p['Wnf1'].shape),
                  _full_spec(p['Wns1'].shape)],
        out_specs=[_row_spec(d) for d in (1, 128, 128, 128, 128, 128)],
        out_shape=t2_out,
    )(cnt, xg0, xf0, xns0, p['Wg1'], p['Wnf1'], p['Wns1'])

    og, ofa, ofb, onsa, onsb = _sc_convs(
        row, col,
        [('edge', zg), ('col', zfa, zfb), ('col', znsa, znsb)], zeros)

    def t3(dinv_ref, og_r, ofa_r, ofb_r, onsa_r, onsb_r,
           bg1, bnf1, bns1, wg2, wnf2,
           xns_o, zg2_o, zf2a_o, zf2b_o):
        dinv_b = dinv_ref[...]
        ogb = og_r[...]
        xg1 = _lr(dinv_b * (ogb[0] + ogb[1]) + bg1[...])
        xf1 = _lr(dinv_b * jnp.concatenate([ofa_r[...], ofb_r[...]], 1)
                  + bnf1[...])
        xns_o[...] = _lr(dinv_b * jnp.concatenate([onsa_r[...], onsb_r[...]], 1)
                         + bns1[...])
        zg2_o[...] = dinv_b * _dot(xg1, wg2[...])
        zf2 = dinv_b * _dot(xf1, wnf2[...])
        zf2a_o[...] = zf2[:, :128]
        zf2b_o[...] = zf2[:, 128:]

    xns, zg2, zf2a, zf2b = _tc_call(
        t3,
        [(dinv, 1), (og, 'part'), (ofa, 128), (ofb, 128),
         (onsa, 128), (onsb, 128),
         (r1(p['bg1']), None), (r1(p['bnf1']), None), (r1(p['bns1']), None),
         (p['Wg2'], None), (p['Wnf2'], None)],
        None, [256, 128, 128, 128])

    og2, of2a, of2b = _sc_convs(
        row, col, [('edge', zg2), ('col', zf2a, zf2b)], zeros)

    headv = jnp.zeros((1, 16), jnp.float32)
    headv = headv.at[0, 0].set(p['Wr1'][0, 0])
    headv = headv.at[0, 1].set(p['Wr1'][0, 1])
    headv = headv.at[0, 2].set(p['Wr1'][1, 0])
    headv = headv.at[0, 3].set(p['Wr1'][1, 1])
    headv = headv.at[0, 4].set(p['br1'][0])
    headv = headv.at[0, 5].set(p['br1'][1])
    headv = headv.at[0, 6].set(p['Wr2'][0, 0])
    headv = headv.at[0, 7].set(p['Wr2'][1, 0])
    headv = headv.at[0, 8].set(p['br2'][0])
    w2c = p['W2'][:, 0].reshape(1, -1)
    w4c = p['W4'][:, 0].reshape(1, -1)
    b2c = p['b2'].reshape(1, 1)
    b4c = p['b4'].reshape(1, 1)

    def t4(dinv_ref, og2_r, of2a_r, of2b_r, bg2, bnf2,
           xd_r, xns_r, wf, bf, w1, b1, w2c_r, b2_r, w3, b3, w4c_r, b4_r,
           hv_r, y_o, sci_o, ssi_o, hci_o, hsi_o):
        dinv_b = dinv_ref[...]
        og2b = og2_r[...]
        xg2 = _lr(dinv_b * (og2b[0] + og2b[1]) + bg2[...])
        xf2 = _lr(dinv_b * jnp.concatenate([of2a_r[...], of2b_r[...]], 1)
                  + bnf2[...])
        hci = _lr(_dot(jnp.concatenate([xd_r[...], xg2], 1), wf[...])
                  + bf[...])
        hsi = xf2 * xns_r[...]
        hci_o[...] = hci
        hsi_o[...] = hsi
        a1 = _lr(_dot(hci, w1[...]) + b1[...])
        sci = jax.nn.sigmoid(
            jnp.sum(a1 * w2c_r[...], axis=1, keepdims=True) + b2_r[...])
        a3 = _lr(_dot(hsi, w3[...]) + b3[...])
        ssi = jax.nn.sigmoid(
            jnp.sum(a3 * w4c_r[...], axis=1, keepdims=True) + b4_r[...])
        sci_o[...] = sci
        ssi_o[...] = ssi
        hv = hv_r[...]
        y0 = _lr(sci * hv[0:1, 0:1] + ssi * hv[0:1, 2:3] + hv[0:1, 4:5])
        y1 = _lr(sci * hv[0:1, 1:2] + ssi * hv[0:1, 3:4] + hv[0:1, 5:6])
        y_o[...] = jax.nn.sigmoid(
            y0 * hv[0:1, 6:7] + y1 * hv[0:1, 7:8] + hv[0:1, 8:9])

    y, sci, ssi, hci, hsi = _tc_call(
        t4,
        [(dinv, 1), (og2, 'part'), (of2a, 128), (of2b, 128),
         (r1(p['bg2']), None), (r1(p['bnf2']), None),
         (xd, 128), (xns, 256),
         (p['Wf'], None), (r1(p['bf']), None),
         (p['W1'], None), (r1(p['b1']), None), (w2c, None), (b2c, None),
         (p['W3'], None), (r1(p['b3']), None), (w4c, None), (b4c, None),
         (headv, None)],
        None, [1, 1, 1, 256, 256])

    return (y.reshape(_N), sci, ssi, hci, hsi)


# speedup vs baseline: 1.0320x; 1.0320x over previous
import functools

import jax
import jax.numpy as jnp
from jax import lax
from jax.experimental import pallas as pl
from jax.experimental.pallas import tpu as pltpu
from jax.experimental.pallas import tpu_sc as plsc

_N = 10000
_E = 320000
_K = 128
_NCH = _E // _K
_NS = 16
_NC = 2
_RPT = 632
_RLAST = _N - _RPT * (_NS - 1)


def _per_tile_rows(s, fn):
    @pl.when(s < _NS - 1)
    def _():
        fn(s * _RPT, _RPT)

    @pl.when(s == _NS - 1)
    def _():
        fn(_RPT * (_NS - 1), _RLAST)


def _lr(x):
    return jnp.where(x >= 0, x, 0.01 * x)


def _sc_mesh():
    return plsc.VectorSubcoreMesh(
        core_axis_name="c", subcore_axis_name="s",
        num_cores=_NC, num_subcores=_NS)


def _sc_degree(col, zeros):
    ones = jnp.ones((_K, 128), jnp.float32)

    @functools.partial(
        pl.kernel,
        out_type=jax.ShapeDtypeStruct((_NC, _N, 128), jnp.float32),
        mesh=_sc_mesh(),
        scratch_types=[
            pltpu.VMEM_SHARED((_N, 128), jnp.float32),
            pltpu.VMEM((_K,), jnp.int32),
            pltpu.VMEM((_K,), jnp.int32),
            pltpu.VMEM((_K, 128), jnp.float32),
            pltpu.SemaphoreType.DMA,
            pltpu.SemaphoreType.DMA,
            pltpu.SemaphoreType.DMA,
            pltpu.SemaphoreType.DMA,
        ],
    )
    def k(col_hbm, ones_hbm, zeros_hbm, cnt_out, acc,
          cb0, cb1, onesbuf, isem0, isem1, ssem0, ssem1):
        cb = (cb0, cb1)
        isem = (isem0, isem1)
        ssem = (ssem0, ssem1)
        c = lax.axis_index("c")
        s = lax.axis_index("s")
        w = c * _NS + s
        _per_tile_rows(s, lambda base, sz: pltpu.sync_copy(
            zeros_hbm.at[pl.ds(base, sz)], acc.at[pl.ds(base, sz)]))
        pltpu.sync_copy(ones_hbm, onesbuf)
        plsc.subcore_barrier()
        npipe = 78

        def base_of(i):
            return (w + jnp.minimum(i, npipe - 1) * (_NC * _NS)) * _K

        def issue_idx(i, x):
            pltpu.async_copy(col_hbm.at[pl.ds(base_of(i), _K)],
                             cb[x], isem[x])

        def wait_idx(x):
            pltpu.make_async_copy(col_hbm.at[pl.ds(0, _K)],
                                  cb[x], isem[x]).wait()

        def wait_scatter(x):
            pltpu.make_async_copy(zeros_hbm.at[pl.ds(0, _K)],
                                  onesbuf, ssem[x]).wait()

        def half(i, x, first=False):
            wait_idx(x)
            pltpu.async_copy(onesbuf, acc.at[cb[x]], ssem[x], add=True)
            if not first:
                wait_scatter(1 - x)
            issue_idx(i + 1, 1 - x)

        issue_idx(0, 0)
        half(0, 0, first=True)

        def body(u, carry):
            half(2 * u + 1, 1)
            half(2 * u + 2, 0)
            return carry

        lax.fori_loop(0, (npipe - 2) // 2, body, 0)
        half(npipe - 1, 1)
        wait_scatter(1)
        wait_idx(0)

        @pl.when(w < 4)
        def _():
            base = (w + npipe * (_NC * _NS)) * _K
            pltpu.sync_copy(col_hbm.at[pl.ds(base, _K)], cb0)
            pltpu.sync_copy(onesbuf, acc.at[cb0], add=True)

        plsc.subcore_barrier()
        _per_tile_rows(s, lambda base, sz: pltpu.sync_copy(
            acc.at[pl.ds(base, sz)], cnt_out.at[c, pl.ds(base, sz)]))

    return k(col, ones, zeros)


def _sc_convs(row, col, convs, zeros):
    ins = []
    out_type = []
    for cv in convs:
        if cv[0] == 'col':
            ins += [cv[1], cv[2]]
            out_type += [jax.ShapeDtypeStruct((_N, 128), jnp.float32)] * 2
        else:
            ins.append(cv[1])
            out_type.append(jax.ShapeDtypeStruct((_NC, _N, 128), jnp.float32))
    n_in = len(ins)
    scratch = [pltpu.VMEM_SHARED((_N, 128), jnp.float32)]
    scratch += [pltpu.VMEM((_K, 128), jnp.float32) for _ in range(2)]
    scratch += [pltpu.VMEM((_K,), jnp.int32) for _ in range(8)]
    scratch += [pltpu.SemaphoreType.DMA for _ in range(8)]

    @functools.partial(
        pl.kernel, out_type=out_type, mesh=_sc_mesh(),
        scratch_types=scratch,
    )
    def k(row_hbm, col_hbm, zeros_hbm, *rest):
        zs = rest[:n_in]
        outs = rest[n_in:2 * n_in]
        sc = rest[2 * n_in:]
        acc = sc[0]
        gb = sc[1:3]
        rb = sc[3:7]
        cb = sc[7:11]
        gsem = sc[11:13]
        ssem = sc[13:15]
        isem = sc[15:19]
        c = lax.axis_index("c")
        s = lax.axis_index("s")

        def edge_sweep(z, chunk0, npipe, nepi):
            def base_of(i):
                return (chunk0 + s + jnp.minimum(i, npipe - 1) * _NS) * _K

            def issue_idx(i, q):
                pltpu.async_copy(row_hbm.at[pl.ds(base_of(i), _K)],
                                 rb[q], isem[q])
                pltpu.async_copy(col_hbm.at[pl.ds(base_of(i), _K)],
                                 cb[q], isem[q])

            def wait_idx(q):
                pltpu.make_async_copy(row_hbm.at[pl.ds(0, _K)],
                                      rb[q], isem[q]).wait()
                pltpu.make_async_copy(col_hbm.at[pl.ds(0, _K)],
                                      cb[q], isem[q]).wait()

            def issue_gather(x, q):
                pltpu.async_copy(z.at[rb[q]], gb[x], gsem[x])

            def wait_gather(x):
                pltpu.make_async_copy(zeros_hbm.at[pl.ds(0, _K)],
                                      gb[x], gsem[x]).wait()

            def issue_scatter(x, q):
                pltpu.async_copy(gb[x], acc.at[cb[q]], ssem[x], add=True)

            def wait_scatter(x):
                pltpu.make_async_copy(zeros_hbm.at[pl.ds(0, _K)],
                                      gb[x], ssem[x]).wait()

            def half(i, p2, p4, first=False):
                wait_gather(p2)
                issue_scatter(p2, p4)
                if not first:
                    wait_scatter(1 - p2)
                wait_idx((p4 + 1) % 4)
                issue_gather(1 - p2, (p4 + 1) % 4)
                issue_idx(i + 2, (p4 + 2) % 4)

            issue_idx(0, 0)
            issue_idx(1, 1)
            wait_idx(0)
            issue_gather(0, 0)
            half(0, 0, 0, first=True)

            w_loop = (npipe - 1) // 4
            rem = npipe - 1 - 4 * w_loop

            def body(u, carry):
                i0 = 4 * u + 1
                for kk in range(4):
                    half(i0 + kk, (1 + kk) % 2, (1 + kk) % 4)
                return carry

            lax.fori_loop(0, w_loop, body, 0)
            for kk in range(rem):
                half(4 * w_loop + 1 + kk, (1 + kk) % 2, (1 + kk) % 4)
            wait_scatter((npipe - 1) % 2)
            wait_gather(npipe % 2)
            wait_idx((npipe + 1) % 4)

            def step(i, carry):
                base = (chunk0 + s + i * _NS) * _K
                pltpu.sync_copy(row_hbm.at[pl.ds(base, _K)], rb[0])
                pltpu.sync_copy(col_hbm.at[pl.ds(base, _K)], cb[0])
                pltpu.async_copy(z.at[rb[0]], gb[0], gsem[0]).wait()
                pltpu.sync_copy(gb[0], acc.at[cb[0]], add=True)
                return carry

            lax.fori_loop(npipe, npipe + nepi, step, 0)

        def init_from(src):
            _per_tile_rows(s, lambda base, sz: pltpu.sync_copy(
                src.at[pl.ds(base, sz)], acc.at[pl.ds(base, sz)]))

        zi = 0
        oi = 0
        for cv in convs:
            if cv[0] == 'col':
                z_lo, z_hi = zs[zi], zs[zi + 1]
                o_lo, o_hi = outs[oi], outs[oi + 1]
                zi += 2
                oi += 2
                nepi = (s < 4).astype(jnp.int32)
                for half, (z, o) in enumerate(((z_lo, o_lo), (z_hi, o_hi))):
                    @pl.when(c == half)
                    def _(z=z, o=o, nepi=nepi):
                        init_from(z)
                        plsc.subcore_barrier()
                        edge_sweep(z, 0, 156, nepi)
                        plsc.subcore_barrier()
                        _per_tile_rows(s, lambda base, sz: pltpu.sync_copy(
                            acc.at[pl.ds(base, sz)], o.at[pl.ds(base, sz)]))
                        plsc.subcore_barrier()
            else:
                z = zs[zi]
                o = outs[oi]
                zi += 1
                oi += 1

                @pl.when(c == 0)
                def _(z=z):
                    init_from(z)

                @pl.when(c == 1)
                def _():
                    init_from(zeros_hbm)

                plsc.subcore_barrier()
                nepi = (s < 2).astype(jnp.int32)
                edge_sweep(z, c * (_NCH // _NC), 78, nepi)
                plsc.subcore_barrier()
                _per_tile_rows(s, lambda base, sz: pltpu.sync_copy(
                    acc.at[pl.ds(base, sz)], o.at[c, pl.ds(base, sz)]))
                plsc.subcore_barrier()

    return k(row, col, zeros, *ins)


_B = 1000
_G = _N // _B


def _row_spec(d):
    return pl.BlockSpec((_B, d), lambda i: (i, 0))


def _part_spec():
    return pl.BlockSpec((_NC, _B, 128), lambda i: (0, i, 0))


def _full_spec(shape):
    nd = len(shape)
    return pl.BlockSpec(shape, lambda i: (0,) * nd)


def _tc_call(body, ins, in_full, outs):
    in_specs = []
    args = []
    for a, d in ins:
        args.append(a)
        if d == 'part':
            in_specs.append(_part_spec())
        elif d is not None:
            in_specs.append(_row_spec(d))
        else:
            in_specs.append(_full_spec(a.shape))
    out_specs = [_row_spec(d) for d in outs]
    out_shape = [jax.ShapeDtypeStruct((_N, d), jnp.float32) for d in outs]
    return pl.pallas_call(
        body,
        grid=(_G,),
        in_specs=in_specs,
        out_specs=out_specs,
        out_shape=out_shape,
    )(*args)


def _dot(a, b):
    return jnp.dot(a, b, preferred_element_type=jnp.float32)


def kernel(discrete_x, continous_x, edge_index, edge_attr, churn_date, params):
    p = params
    row = edge_index[0]
    col = edge_index[1]
    x = discrete_x

    cpad = jnp.pad(churn_date, ((0, 0), (0, 112)))
    wns0 = jnp.pad(p['Wns0'], ((0, 112), (0, 0)))

    def r1(b):
        return b.reshape(1, -1)

    zeros = jnp.zeros((_N, 128), jnp.float32)
    cnt = _sc_degree(col, zeros)

    def t1(x_ref, c_ref, wd, bd, wg0, bg0, wnf0, bnf0, wns0_, bns0,
           xd_o, xg0_o, xf0_o, xns0_o):
        xb = x_ref[...]
        cb = c_ref[...]
        xd_o[...] = _lr(_dot(xb, wd[...]) + bd[...])
        xg0_o[...] = _lr(_dot(xb, wg0[...]) + bg0[...])
        xf0_o[...] = _lr(_dot(xb, wnf0[...]) + bnf0[...])
        xns0_o[...] = _lr(_dot(cb, wns0_[...]) + bns0[...])

    xd, xg0, xf0, xns0 = _tc_call(
        t1,
        [(x, 128), (cpad, 128),
         (p['Wd'], None), (r1(p['bd']), None),
         (p['Wg0'], None), (r1(p['bg0']), None),
         (p['Wnf0'], None), (r1(p['bnf0']), None),
         (wns0, None), (r1(p['bns0']), None)],
        None, [128, 128, 256, 128])

    def t2(cnt_ref, xg0_ref, xf0_ref, xns0_ref, wg1, wnf1, wns1,
           dinv_o, zg_o, zfa_o, zfb_o, znsa_o, znsb_o):
        cb = cnt_ref[...]
        deg = cb[0, :, 0:1] + cb[1, :, 0:1] + 1.0
        dinv = lax.rsqrt(deg)
        dinv_o[...] = dinv
        zg_o[...] = dinv * _dot(xg0_ref[...], wg1[...])
        zf = dinv * _dot(xf0_ref[...], wnf1[...])
        zns = dinv * _dot(xns0_ref[...], wns1[...])
        zfa_o[...] = zf[:, :128]
        zfb_o[...] = zf[:, 128:]
        znsa_o[...] = zns[:, :128]
        znsb_o[...] = zns[:, 128:]

    t2_out = [jax.ShapeDtypeStruct((_N, d), jnp.float32)
              for d in (1, 128, 128, 128, 128, 128)]
    dinv, zg, zfa, zfb, znsa, znsb = pl.pallas_call(
        t2,
        grid=(_G,),
        in_specs=[_part_spec(),
                  _row_spec(128), _row_spec(256), _row_spec(128),
                  _full_spec(p['Wg1'].shape), _full_spec(p['Wnf1'].shape),
                  _full_spec(p['Wns1'].shape)],
        out_specs=[_row_spec(d) for d in (1, 128, 128, 128, 128, 128)],
        out_shape=t2_out,
    )(cnt, xg0, xf0, xns0, p['Wg1'], p['Wnf1'], p['Wns1'])

    og, ofa, ofb, onsa, onsb = _sc_convs(
        row, col,
        [('edge', zg), ('col', zfa, zfb), ('col', znsa, znsb)], zeros)

    def t3(dinv_ref, og_r, ofa_r, ofb_r, onsa_r, onsb_r,
           bg1, bnf1, bns1, wg2, wnf2,
           xns_o, zg2_o, zf2a_o, zf2b_o):
        dinv_b = dinv_ref[...]
        ogb = og_r[...]
        xg1 = _lr(dinv_b * (ogb[0] + ogb[1]) + bg1[...])
        xf1 = _lr(dinv_b * jnp.concatenate([ofa_r[...], ofb_r[...]], 1)
                  + bnf1[...])
        xns_o[...] = _lr(dinv_b * jnp.concatenate([onsa_r[...], onsb_r[...]], 1)
                         + bns1[...])
        zg2_o[...] = dinv_b * _dot(xg1, wg2[...])
        zf2 = dinv_b * _dot(xf1, wnf2[...])
        zf2a_o[...] = zf2[:, :128]
        zf2b_o[...] = zf2[:, 128:]

    xns, zg2, zf2a, zf2b = _tc_call(
        t3,
        [(dinv, 1), (og, 'part'), (ofa, 128), (ofb, 128),
         (onsa, 128), (onsb, 128),
         (r1(p['bg1']), None), (r1(p['bnf1']), None), (r1(p['bns1']), None),
         (p['Wg2'], None), (p['Wnf2'], None)],
        None, [256, 128, 128, 128])

    og2, of2a, of2b = _sc_convs(
        row, col, [('edge', zg2), ('col', zf2a, zf2b)], zeros)

    headv = jnp.zeros((1, 16), jnp.float32)
    headv = headv.at[0, 0].set(p['Wr1'][0, 0])
    headv = headv.at[0, 1].set(p['Wr1'][0, 1])
    headv = headv.at[0, 2].set(p['Wr1'][1, 0])
    headv = headv.at[0, 3].set(p['Wr1'][1, 1])
    headv = headv.at[0, 4].set(p['br1'][0])
    headv = headv.at[0, 5].set(p['br1'][1])
    headv = headv.at[0, 6].set(p['Wr2'][0, 0])
    headv = headv.at[0, 7].set(p['Wr2'][1, 0])
    headv = headv.at[0, 8].set(p['br2'][0])
    w2c = p['W2'][:, 0].reshape(1, -1)
    w4c = p['W4'][:, 0].reshape(1, -1)
    b2c = p['b2'].reshape(1, 1)
    b4c = p['b4'].reshape(1, 1)

    def t4(dinv_ref, og2_r, of2a_r, of2b_r, bg2, bnf2,
           xd_r, xns_r, wf, bf, w1, b1, w2c_r, b2_r, w3, b3, w4c_r, b4_r,
           hv_r, y_o, sci_o, ssi_o, hci_o, hsi_o):
        dinv_b = dinv_ref[...]
        og2b = og2_r[...]
        xg2 = _lr(dinv_b * (og2b[0] + og2b[1]) + bg2[...])
        xf2 = _lr(dinv_b * jnp.concatenate([of2a_r[...], of2b_r[...]], 1)
                  + bnf2[...])
        hci = _lr(_dot(jnp.concatenate([xd_r[...], xg2], 1), wf[...])
                  + bf[...])
        hsi = xf2 * xns_r[...]
        hci_o[...] = hci
        hsi_o[...] = hsi
        a1 = _lr(_dot(hci, w1[...]) + b1[...])
        sci = jax.nn.sigmoid(
            jnp.sum(a1 * w2c_r[...], axis=1, keepdims=True) + b2_r[...])
        a3 = _lr(_dot(hsi, w3[...]) + b3[...])
        ssi = jax.nn.sigmoid(
            jnp.sum(a3 * w4c_r[...], axis=1, keepdims=True) + b4_r[...])
        sci_o[...] = sci
        ssi_o[...] = ssi
        hv = hv_r[...]
        y0 = _lr(sci * hv[0:1, 0:1] + ssi * hv[0:1, 2:3] + hv[0:1, 4:5])
        y1 = _lr(sci * hv[0:1, 1:2] + ssi * hv[0:1, 3:4] + hv[0:1, 5:6])
        y_o[...] = jax.nn.sigmoid(
            y0 * hv[0:1, 6:7] + y1 * hv[0:1, 7:8] + hv[0:1, 8:9])

    y, sci, ssi, hci, hsi = _tc_call(
        t4,
        [(dinv, 1), (og2, 'part'), (of2a, 128), (of2b, 128),
         (r1(p['bg2']), None), (r1(p['bnf2']), None),
         (xd, 128), (xns, 256),
         (p['Wf'], None), (r1(p['bf']), None),
         (p['W1'], None), (r1(p['b1']), None), (w2c, None), (b2c, None),
         (p['W3'], None), (r1(p['b3']), None), (w4c, None), (b4c, None),
         (headv, None)],
        None, [1, 1, 1, 256, 256])

    return (y.reshape(_N), sci, ssi, hci, hsi)
